# Initial kernel scaffold; baseline (speedup 1.0000x reference)
import jax
import jax.numpy as jnp
from jax.experimental import pallas as pl


def kernel(feats, edge_index, W0, b0, W1, b1, W2, b2):
    raise NotImplementedError("write your pallas kernel here")


# speedup vs baseline: 3.2147x; 3.2147x over previous
import functools

import jax
import jax.numpy as jnp
from jax import lax
from jax.experimental import pallas as pl
from jax.experimental.pallas import tpu as pltpu
from jax.experimental.pallas import tpu_sc as plsc

N_NODES = 10000
N_SLAB = 10240
N_EDGES = 320000
IN_DIM = 128
HID = 256

NC = 2
NS = 16
NW = 10
ROWS_PER_W = N_NODES // NW
CHUNK = 80

_MESH = plsc.VectorSubcoreMesh(core_axis_name="c", subcore_axis_name="s")
_SC_PARAMS = pltpu.CompilerParams(use_tc_tiling_on_sc=False)


def _hist_body(src_hbm, dst_hbm, ones_hbm, zeros_hbm, out_hbm,
               idx_v, ones_v, zbuf_v, slab):
    c = lax.axis_index("c")
    s = lax.axis_index("s")

    pltpu.sync_copy(ones_hbm, ones_v)

    @pl.when(s < NW)
    def _():
        pltpu.sync_copy(zeros_hbm, zbuf_v)
        for k in range(5):
            pltpu.sync_copy(zbuf_v, slab.at[pl.ds(s * ROWS_PER_W + k * 200, 200)])

    plsc.subcore_barrier()

    n_e = N_EDGES // NS

    def run(e_ref):
        @pl.loop(0, n_e, step=CHUNK)
        def _(i):
            pltpu.sync_copy(e_ref.at[pl.ds(s * n_e + i, CHUNK)], idx_v)
            pltpu.sync_copy(ones_v, slab.at[idx_v], add=True)

    @pl.when(c == 0)
    def _():
        run(src_hbm)

    @pl.when(c == 1)
    def _():
        run(dst_hbm)

    plsc.subcore_barrier()

    @pl.when(s < NW)
    def _():
        for k in range(5):
            pltpu.sync_copy(
                slab.at[pl.ds(s * ROWS_PER_W + k * 200, 200)],
                out_hbm.at[pl.ds(c * N_NODES + s * ROWS_PER_W + k * 200, 200)],
            )


def _hist(src, dst):
    k = pl.kernel(
        _hist_body,
        out_type=jax.ShapeDtypeStruct((2 * N_NODES, 16), jnp.float32),
        mesh=_MESH,
        compiler_params=_SC_PARAMS,
        scratch_types=[
            pltpu.VMEM((CHUNK,), jnp.int32),
            pltpu.VMEM((CHUNK, 16), jnp.float32),
            pltpu.VMEM((200, 16), jnp.float32),
            pltpu.VMEM_SHARED((N_SLAB, 16), jnp.float32),
        ],
    )
    ones = jnp.ones((CHUNK, 16), jnp.float32)
    zeros = jnp.zeros((200, 16), jnp.float32)
    return k(src, dst, ones, zeros)


def _spmm_body(edge_split, t0, t1, src_hbm, dst_hbm, zeros_hbm, out_hbm,
               src_v, dst_v, rows_v, zbuf_v, slab):
    c = lax.axis_index("c")
    s = lax.axis_index("s")

    @pl.when(s < NW)
    def _():
        pltpu.sync_copy(zeros_hbm, zbuf_v)
        for k in range(5):
            pltpu.sync_copy(zbuf_v, slab.at[pl.ds(s * ROWS_PER_W + k * 200, 200)])

    plsc.subcore_barrier()

    if edge_split:
        n_e = N_EDGES // (NC * NS)
        base_e = c * (N_EDGES // 2) + s * n_e
    else:
        n_e = N_EDGES // NS
        base_e = s * n_e

    def run(tbl):
        @pl.loop(0, n_e, step=CHUNK)
        def _(i):
            pltpu.sync_copy(src_hbm.at[pl.ds(base_e + i, CHUNK)], src_v)
            pltpu.sync_copy(dst_hbm.at[pl.ds(base_e + i, CHUNK)], dst_v)
            pltpu.sync_copy(tbl.at[src_v], rows_v)
            pltpu.sync_copy(rows_v, slab.at[dst_v], add=True)

    @pl.when(c == 0)
    def _():
        run(t0)

    @pl.when(c == 1)
    def _():
        run(t1)

    plsc.subcore_barrier()

    @pl.when(s < NW)
    def _():
        pltpu.sync_copy(
            slab.at[pl.ds(s * ROWS_PER_W, ROWS_PER_W)],
            out_hbm.at[pl.ds(c * N_NODES + s * ROWS_PER_W, ROWS_PER_W)],
        )


def _spmm(t0, t1, src, dst, edge_split):
    k = pl.kernel(
        functools.partial(_spmm_body, edge_split),
        out_type=jax.ShapeDtypeStruct((2 * N_NODES, IN_DIM), jnp.float32),
        mesh=_MESH,
        compiler_params=_SC_PARAMS,
        scratch_types=[
            pltpu.VMEM((CHUNK,), jnp.int32),
            pltpu.VMEM((CHUNK,), jnp.int32),
            pltpu.VMEM((CHUNK, IN_DIM), jnp.float32),
            pltpu.VMEM((200, IN_DIM), jnp.float32),
            pltpu.VMEM_SHARED((N_SLAB, IN_DIM), jnp.float32),
        ],
    )
    zeros = jnp.zeros((200, IN_DIM), jnp.float32)
    return k(t0, t1, src, dst, zeros)


_ROWS_BLK = 1000
_GRID = N_NODES // _ROWS_BLK


def _prep_body(feats_ref, hsrc_ref, x0_ref):
    dn = lax.rsqrt(jnp.maximum(hsrc_ref[:, 0:1], 1.0))
    x0_ref[...] = feats_ref[...] * dn


def _prep(feats, hist):
    return pl.pallas_call(
        _prep_body,
        grid=(_GRID,),
        in_specs=[
            pl.BlockSpec((_ROWS_BLK, IN_DIM), lambda i: (i, 0)),
            pl.BlockSpec((_ROWS_BLK, 16), lambda i: (i, 0)),
        ],
        out_specs=pl.BlockSpec((_ROWS_BLK, IN_DIM), lambda i: (i, 0)),
        out_shape=jax.ShapeDtypeStruct((N_NODES, IN_DIM), jnp.float32),
    )(feats, hist)


def _layer_body(edge_split, a0_ref, a1_ref, hdst_ref, hsrc_ref, w_ref, b_ref,
                o0_ref, o1_ref):
    if edge_split:
        a = a0_ref[...] + a1_ref[...]
    else:
        a = jnp.concatenate([a0_ref[...], a1_ref[...]], axis=1)
    a = a * lax.rsqrt(jnp.maximum(hdst_ref[:, 0:1], 1.0))
    y = jnp.dot(a, w_ref[...], preferred_element_type=jnp.float32) + b_ref[...]
    y = jnp.maximum(y, 0.0) * lax.rsqrt(jnp.maximum(hsrc_ref[:, 0:1], 1.0))
    o0_ref[...] = y[:, :IN_DIM]
    o1_ref[...] = y[:, IN_DIM:]


def _layer(agg, hist, w, b, edge_split):
    in_dim = w.shape[0]
    k = pl.pallas_call(
        functools.partial(_layer_body, edge_split),
        grid=(_GRID,),
        in_specs=[
            pl.BlockSpec((_ROWS_BLK, IN_DIM), lambda i: (i, 0)),
            pl.BlockSpec((_ROWS_BLK, IN_DIM), lambda i: (i + _GRID, 0)),
            pl.BlockSpec((_ROWS_BLK, 16), lambda i: (i + _GRID, 0)),
            pl.BlockSpec((_ROWS_BLK, 16), lambda i: (i, 0)),
            pl.BlockSpec((in_dim, HID), lambda i: (0, 0)),
            pl.BlockSpec((1, HID), lambda i: (0, 0)),
        ],
        out_specs=[
            pl.BlockSpec((_ROWS_BLK, IN_DIM), lambda i: (i, 0)),
            pl.BlockSpec((_ROWS_BLK, IN_DIM), lambda i: (i, 0)),
        ],
        out_shape=[
            jax.ShapeDtypeStruct((N_NODES, IN_DIM), jnp.float32),
            jax.ShapeDtypeStruct((N_NODES, IN_DIM), jnp.float32),
        ],
    )
    return k(agg, agg, hist, hist, w, b.reshape(1, HID))


def _final_body(a0_ref, a1_ref, hdst_ref, w_ref, b_ref, out_ref):
    i = pl.program_id(0)
    a = jnp.concatenate([a0_ref[...], a1_ref[...]], axis=1)
    a = a * lax.rsqrt(jnp.maximum(hdst_ref[:, 0:1], 1.0))
    y = jnp.dot(a, w_ref[...], preferred_element_type=jnp.float32) + b_ref[...]
    y = jnp.maximum(y, 0.0)
    blk = jnp.sum(y, axis=0, keepdims=True) * (1.0 / N_NODES)

    @pl.when(i == 0)
    def _():
        out_ref[...] = blk

    @pl.when(i > 0)
    def _():
        out_ref[...] += blk


def _final(agg, hist, w, b):
    return pl.pallas_call(
        _final_body,
        grid=(_GRID,),
        in_specs=[
            pl.BlockSpec((_ROWS_BLK, IN_DIM), lambda i: (i, 0)),
            pl.BlockSpec((_ROWS_BLK, IN_DIM), lambda i: (i + _GRID, 0)),
            pl.BlockSpec((_ROWS_BLK, 16), lambda i: (i + _GRID, 0)),
            pl.BlockSpec((HID, HID), lambda i: (0, 0)),
            pl.BlockSpec((1, HID), lambda i: (0, 0)),
        ],
        out_specs=pl.BlockSpec((1, HID), lambda i: (0, 0)),
        out_shape=jax.ShapeDtypeStruct((1, HID), jnp.float32),
    )(agg, agg, hist, w, b.reshape(1, HID))


def kernel(feats, edge_index, W0, b0, W1, b1, W2, b2):
    src = edge_index[0].astype(jnp.int32)
    dst = edge_index[1].astype(jnp.int32)

    hist = _hist(src, dst)
    x0 = _prep(feats, hist)
    agg0 = _spmm(x0, x0, src, dst, edge_split=True)
    h1a, h1b = _layer(agg0, hist, W0, b0, edge_split=True)
    agg1 = _spmm(h1a, h1b, src, dst, edge_split=False)
    h2a, h2b = _layer(agg1, hist, W1, b1, edge_split=False)
    agg2 = _spmm(h2a, h2b, src, dst, edge_split=False)
    return _final(agg2, hist, W2, b2)


# speedup vs baseline: 5.4159x; 1.6847x over previous
import functools

import jax
import jax.numpy as jnp
from jax import lax
from jax.experimental import pallas as pl
from jax.experimental.pallas import tpu as pltpu
from jax.experimental.pallas import tpu_sc as plsc

N_NODES = 10000
N_SLAB = 10240
N_EDGES = 320000
IN_DIM = 128
HID = 256

NC = 2
NS = 16
NW = 10
ROWS_PER_W = N_NODES // NW
CHUNK = 80
NBLK = 2000

_MESH = plsc.VectorSubcoreMesh(core_axis_name="c", subcore_axis_name="s")
_SC_PARAMS = pltpu.CompilerParams(use_tc_tiling_on_sc=False)


def _hist_body(src_hbm, dst_hbm, ones_hbm, zeros_hbm, out_hbm,
               idx_v, ones_v, zbuf_v, slab):
    c = lax.axis_index("c")
    s = lax.axis_index("s")

    pltpu.sync_copy(ones_hbm, ones_v)

    @pl.when(s < NW)
    def _():
        pltpu.sync_copy(zeros_hbm, zbuf_v)
        for k in range(5):
            pltpu.sync_copy(zbuf_v, slab.at[pl.ds(s * ROWS_PER_W + k * 200, 200)])

    plsc.subcore_barrier()

    n_e = N_EDGES // NS

    def run(e_ref):
        @pl.loop(0, n_e, step=CHUNK)
        def _(i):
            pltpu.sync_copy(e_ref.at[pl.ds(s * n_e + i, CHUNK)], idx_v)
            pltpu.sync_copy(ones_v, slab.at[idx_v], add=True)

    @pl.when(c == 0)
    def _():
        run(src_hbm)

    @pl.when(c == 1)
    def _():
        run(dst_hbm)

    plsc.subcore_barrier()

    @pl.when(s < NW)
    def _():
        for k in range(5):
            pltpu.sync_copy(
                slab.at[pl.ds(s * ROWS_PER_W + k * 200, 200)],
                out_hbm.at[pl.ds(c * N_NODES + s * ROWS_PER_W + k * 200, 200)],
            )


def _hist(src, dst):
    k = pl.kernel(
        _hist_body,
        out_type=jax.ShapeDtypeStruct((2 * N_NODES, 16), jnp.float32),
        mesh=_MESH,
        compiler_params=_SC_PARAMS,
        scratch_types=[
            pltpu.VMEM((CHUNK,), jnp.int32),
            pltpu.VMEM((CHUNK, 16), jnp.float32),
            pltpu.VMEM((200, 16), jnp.float32),
            pltpu.VMEM_SHARED((N_SLAB, 16), jnp.float32),
        ],
    )
    ones = jnp.ones((CHUNK, 16), jnp.float32)
    zeros = jnp.zeros((200, 16), jnp.float32)
    return k(src, dst, ones, zeros)


def _spmm_body(edge_split, t0, t1, src_hbm, dst_hbm, zeros_hbm, out_hbm,
               sidx, didx, rows0, rows1, zbuf_v, slab, sem0, sem1):
    c = lax.axis_index("c")
    s = lax.axis_index("s")

    if edge_split:
        n_e = N_EDGES // (NC * NS)
        base_e = c * (N_EDGES // 2) + s * n_e
    else:
        n_e = N_EDGES // NS
        base_e = s * n_e

    @pl.when(s < NW)
    def _():
        pltpu.sync_copy(zeros_hbm, zbuf_v)
        for k in range(8):
            pltpu.sync_copy(zbuf_v, slab.at[pl.ds(s * ROWS_PER_W + k * 125, 125)])

    plsc.subcore_barrier()

    def gidx(j):
        return sidx.at[pl.ds(j, CHUNK)]

    def widx(j):
        return didx.at[pl.ds(j, CHUNK)]

    def run(tbl):
        @pl.loop(0, n_e, step=NBLK)
        def _(i):
            pltpu.sync_copy(src_hbm.at[pl.ds(base_e + i, NBLK)], sidx)
            pltpu.sync_copy(dst_hbm.at[pl.ds(base_e + i, NBLK)], didx)

            @pl.loop(0, (NBLK // CHUNK // 2) * 2 * CHUNK, step=2 * CHUNK)
            def _(j0):
                g0 = pltpu.async_copy(tbl.at[gidx(j0)], rows0, sem0)
                g1 = pltpu.async_copy(tbl.at[gidx(j0 + CHUNK)], rows1, sem1)
                g0.wait()
                pltpu.sync_copy(rows0, slab.at[widx(j0)], add=True)
                g1.wait()
                pltpu.sync_copy(rows1, slab.at[widx(j0 + CHUNK)], add=True)

            if (NBLK // CHUNK) % 2:
                j = NBLK - CHUNK
                pltpu.async_copy(tbl.at[gidx(j)], rows0, sem0).wait()
                pltpu.sync_copy(rows0, slab.at[widx(j)], add=True)

    @pl.when(c == 0)
    def _():
        run(t0)

    @pl.when(c == 1)
    def _():
        run(t1)

    plsc.subcore_barrier()

    @pl.when(s < NW)
    def _():
        pltpu.sync_copy(
            slab.at[pl.ds(s * ROWS_PER_W, ROWS_PER_W)],
            out_hbm.at[pl.ds(c * N_NODES + s * ROWS_PER_W, ROWS_PER_W)],
        )


def _spmm(t0, t1, src, dst, edge_split):
    k = pl.kernel(
        functools.partial(_spmm_body, edge_split),
        out_type=jax.ShapeDtypeStruct((2 * N_NODES, IN_DIM), jnp.float32),
        mesh=_MESH,
        compiler_params=_SC_PARAMS,
        scratch_types=[
            pltpu.VMEM((NBLK,), jnp.int32),
            pltpu.VMEM((NBLK,), jnp.int32),
            pltpu.VMEM((CHUNK, IN_DIM), jnp.float32),
            pltpu.VMEM((CHUNK, IN_DIM), jnp.float32),
            pltpu.VMEM((125, IN_DIM), jnp.float32),
            pltpu.VMEM_SHARED((N_SLAB, IN_DIM), jnp.float32),
            pltpu.SemaphoreType.DMA,
            pltpu.SemaphoreType.DMA,
        ],
    )
    zeros = jnp.zeros((125, IN_DIM), jnp.float32)
    return k(t0, t1, src, dst, zeros)


_ROWS_BLK = 1000
_GRID = N_NODES // _ROWS_BLK


def _prep_body(feats_ref, hsrc_ref, x0_ref):
    dn = lax.rsqrt(jnp.maximum(hsrc_ref[:, 0:1], 1.0))
    x0_ref[...] = feats_ref[...] * dn


def _prep(feats, hist):
    return pl.pallas_call(
        _prep_body,
        grid=(_GRID,),
        in_specs=[
            pl.BlockSpec((_ROWS_BLK, IN_DIM), lambda i: (i, 0)),
            pl.BlockSpec((_ROWS_BLK, 16), lambda i: (i, 0)),
        ],
        out_specs=pl.BlockSpec((_ROWS_BLK, IN_DIM), lambda i: (i, 0)),
        out_shape=jax.ShapeDtypeStruct((N_NODES, IN_DIM), jnp.float32),
    )(feats, hist)


def _layer_body(edge_split, a0_ref, a1_ref, hdst_ref, hsrc_ref, w_ref, b_ref,
                o0_ref, o1_ref):
    if edge_split:
        a = a0_ref[...] + a1_ref[...]
    else:
        a = jnp.concatenate([a0_ref[...], a1_ref[...]], axis=1)
    a = a * lax.rsqrt(jnp.maximum(hdst_ref[:, 0:1], 1.0))
    y = jnp.dot(a, w_ref[...], preferred_element_type=jnp.float32) + b_ref[...]
    y = jnp.maximum(y, 0.0) * lax.rsqrt(jnp.maximum(hsrc_ref[:, 0:1], 1.0))
    o0_ref[...] = y[:, :IN_DIM]
    o1_ref[...] = y[:, IN_DIM:]


def _layer(agg, hist, w, b, edge_split):
    in_dim = w.shape[0]
    k = pl.pallas_call(
        functools.partial(_layer_body, edge_split),
        grid=(_GRID,),
        in_specs=[
            pl.BlockSpec((_ROWS_BLK, IN_DIM), lambda i: (i, 0)),
            pl.BlockSpec((_ROWS_BLK, IN_DIM), lambda i: (i + _GRID, 0)),
            pl.BlockSpec((_ROWS_BLK, 16), lambda i: (i + _GRID, 0)),
            pl.BlockSpec((_ROWS_BLK, 16), lambda i: (i, 0)),
            pl.BlockSpec((in_dim, HID), lambda i: (0, 0)),
            pl.BlockSpec((1, HID), lambda i: (0, 0)),
        ],
        out_specs=[
            pl.BlockSpec((_ROWS_BLK, IN_DIM), lambda i: (i, 0)),
            pl.BlockSpec((_ROWS_BLK, IN_DIM), lambda i: (i, 0)),
        ],
        out_shape=[
            jax.ShapeDtypeStruct((N_NODES, IN_DIM), jnp.float32),
            jax.ShapeDtypeStruct((N_NODES, IN_DIM), jnp.float32),
        ],
    )
    return k(agg, agg, hist, hist, w, b.reshape(1, HID))


def _final_body(a0_ref, a1_ref, hdst_ref, w_ref, b_ref, out_ref):
    i = pl.program_id(0)
    a = jnp.concatenate([a0_ref[...], a1_ref[...]], axis=1)
    a = a * lax.rsqrt(jnp.maximum(hdst_ref[:, 0:1], 1.0))
    y = jnp.dot(a, w_ref[...], preferred_element_type=jnp.float32) + b_ref[...]
    y = jnp.maximum(y, 0.0)
    blk = jnp.sum(y, axis=0, keepdims=True) * (1.0 / N_NODES)

    @pl.when(i == 0)
    def _():
        out_ref[...] = blk

    @pl.when(i > 0)
    def _():
        out_ref[...] += blk


def _final(agg, hist, w, b):
    return pl.pallas_call(
        _final_body,
        grid=(_GRID,),
        in_specs=[
            pl.BlockSpec((_ROWS_BLK, IN_DIM), lambda i: (i, 0)),
            pl.BlockSpec((_ROWS_BLK, IN_DIM), lambda i: (i + _GRID, 0)),
            pl.BlockSpec((_ROWS_BLK, 16), lambda i: (i + _GRID, 0)),
            pl.BlockSpec((HID, HID), lambda i: (0, 0)),
            pl.BlockSpec((1, HID), lambda i: (0, 0)),
        ],
        out_specs=pl.BlockSpec((1, HID), lambda i: (0, 0)),
        out_shape=jax.ShapeDtypeStruct((1, HID), jnp.float32),
    )(agg, agg, hist, w, b.reshape(1, HID))


def kernel(feats, edge_index, W0, b0, W1, b1, W2, b2):
    src = edge_index[0].astype(jnp.int32)
    dst = edge_index[1].astype(jnp.int32)

    hist = _hist(src, dst)
    x0 = _prep(feats, hist)
    agg0 = _spmm(x0, x0, src, dst, edge_split=True)
    h1a, h1b = _layer(agg0, hist, W0, b0, edge_split=True)
    agg1 = _spmm(h1a, h1b, src, dst, edge_split=False)
    h2a, h2b = _layer(agg1, hist, W1, b1, edge_split=False)
    agg2 = _spmm(h2a, h2b, src, dst, edge_split=False)
    return _final(agg2, hist, W2, b2)


# speedup vs baseline: 5.9903x; 1.1060x over previous
import functools

import jax
import jax.numpy as jnp
from jax import lax
from jax.experimental import pallas as pl
from jax.experimental.pallas import tpu as pltpu
from jax.experimental.pallas import tpu_sc as plsc

N_NODES = 10000
N_SLAB = 10240
N_EDGES = 320000
IN_DIM = 128
HID = 256

NC = 2
NS = 16
NW = 10
ROWS_PER_W = N_NODES // NW
CHUNK = 80
NBLK = 2000

_MESH = plsc.VectorSubcoreMesh(core_axis_name="c", subcore_axis_name="s")
_SC_PARAMS = pltpu.CompilerParams(use_tc_tiling_on_sc=False)


def _hist_body(src_hbm, dst_hbm, ones_hbm, zeros_hbm, out_hbm,
               idx_v, ones_v, zbuf_v, slab):
    c = lax.axis_index("c")
    s = lax.axis_index("s")

    pltpu.sync_copy(ones_hbm, ones_v)

    @pl.when(s < NW)
    def _():
        pltpu.sync_copy(zeros_hbm, zbuf_v)
        for k in range(5):
            pltpu.sync_copy(zbuf_v, slab.at[pl.ds(s * ROWS_PER_W + k * 200, 200)])

    plsc.subcore_barrier()

    n_e = N_EDGES // NS

    def run(e_ref):
        @pl.loop(0, n_e, step=NBLK)
        def _(i):
            pltpu.sync_copy(e_ref.at[pl.ds(s * n_e + i, NBLK)], idx_v)

            @pl.loop(0, NBLK, step=CHUNK)
            def _(j):
                pltpu.sync_copy(ones_v, slab.at[idx_v.at[pl.ds(j, CHUNK)]],
                                add=True)

    @pl.when(c == 0)
    def _():
        run(src_hbm)

    @pl.when(c == 1)
    def _():
        run(dst_hbm)

    plsc.subcore_barrier()

    @pl.when(s < NW)
    def _():
        for k in range(5):
            pltpu.sync_copy(
                slab.at[pl.ds(s * ROWS_PER_W + k * 200, 200)],
                out_hbm.at[pl.ds(c * N_NODES + s * ROWS_PER_W + k * 200, 200)],
            )


def _hist(src, dst):
    k = pl.kernel(
        _hist_body,
        out_type=jax.ShapeDtypeStruct((2 * N_NODES, 16), jnp.float32),
        mesh=_MESH,
        compiler_params=_SC_PARAMS,
        scratch_types=[
            pltpu.VMEM((NBLK,), jnp.int32),
            pltpu.VMEM((CHUNK, 16), jnp.float32),
            pltpu.VMEM((200, 16), jnp.float32),
            pltpu.VMEM_SHARED((N_SLAB, 16), jnp.float32),
        ],
    )
    ones = jnp.ones((CHUNK, 16), jnp.float32)
    zeros = jnp.zeros((200, 16), jnp.float32)
    return k(src, dst, ones, zeros)


def _spmm_body(edge_split, t0, t1, src_hbm, dst_hbm, zeros_hbm, out_hbm,
               sidx, didx, rows0, rows1, zbuf_v, slab, sem0, sem1):
    c = lax.axis_index("c")
    s = lax.axis_index("s")

    if edge_split:
        n_e = N_EDGES // (NC * NS)
        base_e = c * (N_EDGES // 2) + s * n_e
    else:
        n_e = N_EDGES // NS
        base_e = s * n_e

    @pl.when(s < NW)
    def _():
        pltpu.sync_copy(zeros_hbm, zbuf_v)
        for k in range(8):
            pltpu.sync_copy(zbuf_v, slab.at[pl.ds(s * ROWS_PER_W + k * 125, 125)])

    plsc.subcore_barrier()

    def gidx(j):
        return sidx.at[pl.ds(j, CHUNK)]

    def widx(j):
        return didx.at[pl.ds(j, CHUNK)]

    def run(tbl):
        @pl.loop(0, n_e, step=NBLK)
        def _(i):
            pltpu.sync_copy(src_hbm.at[pl.ds(base_e + i, NBLK)], sidx)
            pltpu.sync_copy(dst_hbm.at[pl.ds(base_e + i, NBLK)], didx)

            @pl.loop(0, (NBLK // CHUNK // 2) * 2 * CHUNK, step=2 * CHUNK)
            def _(j0):
                g0 = pltpu.async_copy(tbl.at[gidx(j0)], rows0, sem0)
                g1 = pltpu.async_copy(tbl.at[gidx(j0 + CHUNK)], rows1, sem1)
                g0.wait()
                pltpu.sync_copy(rows0, slab.at[widx(j0)], add=True)
                g1.wait()
                pltpu.sync_copy(rows1, slab.at[widx(j0 + CHUNK)], add=True)

            if (NBLK // CHUNK) % 2:
                j = NBLK - CHUNK
                pltpu.async_copy(tbl.at[gidx(j)], rows0, sem0).wait()
                pltpu.sync_copy(rows0, slab.at[widx(j)], add=True)

    @pl.when(c == 0)
    def _():
        run(t0)

    @pl.when(c == 1)
    def _():
        run(t1)

    plsc.subcore_barrier()

    @pl.when(s < NW)
    def _():
        pltpu.sync_copy(
            slab.at[pl.ds(s * ROWS_PER_W, ROWS_PER_W)],
            out_hbm.at[pl.ds(c * N_NODES + s * ROWS_PER_W, ROWS_PER_W)],
        )


def _spmm(t0, t1, src, dst, edge_split):
    k = pl.kernel(
        functools.partial(_spmm_body, edge_split),
        out_type=jax.ShapeDtypeStruct((2 * N_NODES, IN_DIM), jnp.float32),
        mesh=_MESH,
        compiler_params=_SC_PARAMS,
        scratch_types=[
            pltpu.VMEM((NBLK,), jnp.int32),
            pltpu.VMEM((NBLK,), jnp.int32),
            pltpu.VMEM((CHUNK, IN_DIM), jnp.float32),
            pltpu.VMEM((CHUNK, IN_DIM), jnp.float32),
            pltpu.VMEM((125, IN_DIM), jnp.float32),
            pltpu.VMEM_SHARED((N_SLAB, IN_DIM), jnp.float32),
            pltpu.SemaphoreType.DMA,
            pltpu.SemaphoreType.DMA,
        ],
    )
    zeros = jnp.zeros((125, IN_DIM), jnp.float32)
    return k(t0, t1, src, dst, zeros)


_ROWS_BLK = 1000
_GRID = N_NODES // _ROWS_BLK


def _prep_body(feats_ref, hsrc_ref, x0_ref):
    dn = lax.rsqrt(jnp.maximum(hsrc_ref[:, 0:1], 1.0))
    x0_ref[...] = feats_ref[...] * dn


def _prep(feats, hist):
    return pl.pallas_call(
        _prep_body,
        grid=(_GRID,),
        in_specs=[
            pl.BlockSpec((_ROWS_BLK, IN_DIM), lambda i: (i, 0)),
            pl.BlockSpec((_ROWS_BLK, 16), lambda i: (i, 0)),
        ],
        out_specs=pl.BlockSpec((_ROWS_BLK, IN_DIM), lambda i: (i, 0)),
        out_shape=jax.ShapeDtypeStruct((N_NODES, IN_DIM), jnp.float32),
    )(feats, hist)


def _layer_body(edge_split, a0_ref, a1_ref, hdst_ref, hsrc_ref, w_ref, b_ref,
                o0_ref, o1_ref):
    if edge_split:
        a = a0_ref[...] + a1_ref[...]
    else:
        a = jnp.concatenate([a0_ref[...], a1_ref[...]], axis=1)
    a = a * lax.rsqrt(jnp.maximum(hdst_ref[:, 0:1], 1.0))
    y = jnp.dot(a, w_ref[...], preferred_element_type=jnp.float32) + b_ref[...]
    y = jnp.maximum(y, 0.0) * lax.rsqrt(jnp.maximum(hsrc_ref[:, 0:1], 1.0))
    o0_ref[...] = y[:, :IN_DIM]
    o1_ref[...] = y[:, IN_DIM:]


def _layer(agg, hist, w, b, edge_split):
    in_dim = w.shape[0]
    k = pl.pallas_call(
        functools.partial(_layer_body, edge_split),
        grid=(_GRID,),
        in_specs=[
            pl.BlockSpec((_ROWS_BLK, IN_DIM), lambda i: (i, 0)),
            pl.BlockSpec((_ROWS_BLK, IN_DIM), lambda i: (i + _GRID, 0)),
            pl.BlockSpec((_ROWS_BLK, 16), lambda i: (i + _GRID, 0)),
            pl.BlockSpec((_ROWS_BLK, 16), lambda i: (i, 0)),
            pl.BlockSpec((in_dim, HID), lambda i: (0, 0)),
            pl.BlockSpec((1, HID), lambda i: (0, 0)),
        ],
        out_specs=[
            pl.BlockSpec((_ROWS_BLK, IN_DIM), lambda i: (i, 0)),
            pl.BlockSpec((_ROWS_BLK, IN_DIM), lambda i: (i, 0)),
        ],
        out_shape=[
            jax.ShapeDtypeStruct((N_NODES, IN_DIM), jnp.float32),
            jax.ShapeDtypeStruct((N_NODES, IN_DIM), jnp.float32),
        ],
    )
    return k(agg, agg, hist, hist, w, b.reshape(1, HID))


def _final_body(a0_ref, a1_ref, hdst_ref, w_ref, b_ref, out_ref):
    i = pl.program_id(0)
    a = jnp.concatenate([a0_ref[...], a1_ref[...]], axis=1)
    a = a * lax.rsqrt(jnp.maximum(hdst_ref[:, 0:1], 1.0))
    y = jnp.dot(a, w_ref[...], preferred_element_type=jnp.float32) + b_ref[...]
    y = jnp.maximum(y, 0.0)
    blk = jnp.sum(y, axis=0, keepdims=True) * (1.0 / N_NODES)

    @pl.when(i == 0)
    def _():
        out_ref[...] = blk

    @pl.when(i > 0)
    def _():
        out_ref[...] += blk


def _final(agg, hist, w, b):
    return pl.pallas_call(
        _final_body,
        grid=(_GRID,),
        in_specs=[
            pl.BlockSpec((_ROWS_BLK, IN_DIM), lambda i: (i, 0)),
            pl.BlockSpec((_ROWS_BLK, IN_DIM), lambda i: (i + _GRID, 0)),
            pl.BlockSpec((_ROWS_BLK, 16), lambda i: (i + _GRID, 0)),
            pl.BlockSpec((HID, HID), lambda i: (0, 0)),
            pl.BlockSpec((1, HID), lambda i: (0, 0)),
        ],
        out_specs=pl.BlockSpec((1, HID), lambda i: (0, 0)),
        out_shape=jax.ShapeDtypeStruct((1, HID), jnp.float32),
    )(agg, agg, hist, w, b.reshape(1, HID))


def kernel(feats, edge_index, W0, b0, W1, b1, W2, b2):
    src = edge_index[0].astype(jnp.int32)
    dst = edge_index[1].astype(jnp.int32)

    hist = _hist(src, dst)
    x0 = _prep(feats, hist)
    agg0 = _spmm(x0, x0, src, dst, edge_split=True)
    h1a, h1b = _layer(agg0, hist, W0, b0, edge_split=True)
    agg1 = _spmm(h1a, h1b, src, dst, edge_split=False)
    h2a, h2b = _layer(agg1, hist, W1, b1, edge_split=False)
    agg2 = _spmm(h2a, h2b, src, dst, edge_split=False)
    return _final(agg2, hist, W2, b2)


# speedup vs baseline: 7.6686x; 1.2802x over previous
import functools

import jax
import jax.numpy as jnp
from jax import lax
from jax.experimental import pallas as pl
from jax.experimental.pallas import tpu as pltpu
from jax.experimental.pallas import tpu_sc as plsc

N_NODES = 10000
N_SLAB = 10240
N_EDGES = 320000
IN_DIM = 128
HID = 256

NC = 2
NS = 16
NW = 10
ROWS_PER_W = N_NODES // NW
CHUNK = 80
NBLK = 2000

_MESH = plsc.VectorSubcoreMesh(core_axis_name="c", subcore_axis_name="s")
_SC_PARAMS = pltpu.CompilerParams(use_tc_tiling_on_sc=False)


def _hist_body(src_hbm, dst_hbm, ones_hbm, zeros_hbm, out_hbm,
               idx_v, ones_v, zbuf_v, slab):
    c = lax.axis_index("c")
    s = lax.axis_index("s")

    pltpu.sync_copy(ones_hbm, ones_v)

    @pl.when(s < NW)
    def _():
        pltpu.sync_copy(zeros_hbm, zbuf_v)
        for k in range(5):
            pltpu.sync_copy(zbuf_v, slab.at[pl.ds(s * ROWS_PER_W + k * 200, 200)])

    plsc.subcore_barrier()

    n_e = N_EDGES // NS

    def run(e_ref):
        @pl.loop(0, n_e, step=NBLK)
        def _(i):
            pltpu.sync_copy(e_ref.at[pl.ds(s * n_e + i, NBLK)], idx_v)

            @pl.loop(0, NBLK, step=CHUNK)
            def _(j):
                pltpu.sync_copy(ones_v, slab.at[idx_v.at[pl.ds(j, CHUNK)]],
                                add=True)

    @pl.when(c == 0)
    def _():
        run(src_hbm)

    @pl.when(c == 1)
    def _():
        run(dst_hbm)

    plsc.subcore_barrier()

    @pl.when(s < NW)
    def _():
        for k in range(5):
            pltpu.sync_copy(
                slab.at[pl.ds(s * ROWS_PER_W + k * 200, 200)],
                out_hbm.at[pl.ds(c * N_NODES + s * ROWS_PER_W + k * 200, 200)],
            )


def _hist(src, dst):
    k = pl.kernel(
        _hist_body,
        out_type=jax.ShapeDtypeStruct((2 * N_NODES, 16), jnp.float32),
        mesh=_MESH,
        compiler_params=_SC_PARAMS,
        scratch_types=[
            pltpu.VMEM((NBLK,), jnp.int32),
            pltpu.VMEM((CHUNK, 16), jnp.float32),
            pltpu.VMEM((200, 16), jnp.float32),
            pltpu.VMEM_SHARED((N_SLAB, 16), jnp.float32),
        ],
    )
    ones = jnp.ones((CHUNK, 16), jnp.float32)
    zeros = jnp.zeros((200, 16), jnp.float32)
    return k(src, dst, ones, zeros)


def _spmm_body(edge_split, t0, t1, src_hbm, dst_hbm, zeros_hbm, out_hbm,
               sidx, didx, r0, r1, r2, r3, g0, g1, g2, g3, s0, s1, s2, s3,
               slab):
    c = lax.axis_index("c")
    s = lax.axis_index("s")
    bufs = (r0, r1, r2, r3)
    gsem = (g0, g1, g2, g3)
    ssem = (s0, s1, s2, s3)

    if edge_split:
        n_e = N_EDGES // (NC * NS)
        base_e = c * (N_EDGES // 2) + s * n_e
    else:
        n_e = N_EDGES // NS
        base_e = s * n_e

    @pl.when(s < NW)
    def _():
        pltpu.sync_copy(zeros_hbm, r0)
        for k in range(12):
            pltpu.sync_copy(r0, slab.at[pl.ds(s * ROWS_PER_W + k * CHUNK, CHUNK)])
        pltpu.sync_copy(r0.at[pl.ds(0, 40)],
                        slab.at[pl.ds(s * ROWS_PER_W + 12 * CHUNK, 40)])

    plsc.subcore_barrier()

    NCH = NBLK // CHUNK
    NQ = NCH // 4

    def gidx(j):
        return sidx.at[pl.ds(j, CHUNK)]

    def widx(j):
        return didx.at[pl.ds(j, CHUNK)]

    def run(tbl):
        def wait_gather(t):
            pltpu.make_async_copy(tbl.at[pl.ds(0, CHUNK)], bufs[t], gsem[t]).wait()

        @pl.loop(0, n_e, step=NBLK)
        def _(i):
            pltpu.sync_copy(src_hbm.at[pl.ds(base_e + i, NBLK)], sidx)
            pltpu.sync_copy(dst_hbm.at[pl.ds(base_e + i, NBLK)], didx)

            for t in range(4):
                pltpu.async_copy(tbl.at[gidx(t * CHUNK)], bufs[t], gsem[t])

            @pl.loop(0, NQ - 1)
            def _(q):
                jc = q * 4 * CHUNK
                jn = jc + 4 * CHUNK
                sc = []
                for t in range(4):
                    wait_gather(t)
                    sc.append(pltpu.async_copy(
                        bufs[t], slab.at[widx(jc + t * CHUNK)], ssem[t],
                        add=True))
                for t in range(4):
                    sc[t].wait()
                    pltpu.async_copy(tbl.at[gidx(jn + t * CHUNK)], bufs[t],
                                     gsem[t])

            jl = (NQ - 1) * 4 * CHUNK
            for t in range(4):
                wait_gather(t)
                pltpu.sync_copy(bufs[t], slab.at[widx(jl + t * CHUNK)], add=True)
            jt = NQ * 4 * CHUNK
            pltpu.async_copy(tbl.at[gidx(jt)], r0, g0).wait()
            pltpu.sync_copy(r0, slab.at[widx(jt)], add=True)

    @pl.when(c == 0)
    def _():
        run(t0)

    @pl.when(c == 1)
    def _():
        run(t1)

    plsc.subcore_barrier()

    @pl.when(s < NW)
    def _():
        pltpu.sync_copy(
            slab.at[pl.ds(s * ROWS_PER_W, ROWS_PER_W)],
            out_hbm.at[pl.ds(c * N_NODES + s * ROWS_PER_W, ROWS_PER_W)],
        )


def _spmm(t0, t1, src, dst, edge_split):
    k = pl.kernel(
        functools.partial(_spmm_body, edge_split),
        out_type=jax.ShapeDtypeStruct((2 * N_NODES, IN_DIM), jnp.float32),
        mesh=_MESH,
        compiler_params=_SC_PARAMS,
        scratch_types=[
            pltpu.VMEM((NBLK,), jnp.int32),
            pltpu.VMEM((NBLK,), jnp.int32),
            pltpu.VMEM((CHUNK, IN_DIM), jnp.float32),
            pltpu.VMEM((CHUNK, IN_DIM), jnp.float32),
            pltpu.VMEM((CHUNK, IN_DIM), jnp.float32),
            pltpu.VMEM((CHUNK, IN_DIM), jnp.float32),
            pltpu.SemaphoreType.DMA,
            pltpu.SemaphoreType.DMA,
            pltpu.SemaphoreType.DMA,
            pltpu.SemaphoreType.DMA,
            pltpu.SemaphoreType.DMA,
            pltpu.SemaphoreType.DMA,
            pltpu.SemaphoreType.DMA,
            pltpu.SemaphoreType.DMA,
            pltpu.VMEM_SHARED((N_SLAB, IN_DIM), jnp.float32),
        ],
    )
    zeros = jnp.zeros((CHUNK, IN_DIM), jnp.float32)
    return k(t0, t1, src, dst, zeros)


_ROWS_BLK = 1000
_GRID = N_NODES // _ROWS_BLK


def _prep_body(feats_ref, hsrc_ref, x0_ref):
    dn = lax.rsqrt(jnp.maximum(hsrc_ref[:, 0:1], 1.0))
    x0_ref[...] = feats_ref[...] * dn


def _prep(feats, hist):
    return pl.pallas_call(
        _prep_body,
        grid=(_GRID,),
        in_specs=[
            pl.BlockSpec((_ROWS_BLK, IN_DIM), lambda i: (i, 0)),
            pl.BlockSpec((_ROWS_BLK, 16), lambda i: (i, 0)),
        ],
        out_specs=pl.BlockSpec((_ROWS_BLK, IN_DIM), lambda i: (i, 0)),
        out_shape=jax.ShapeDtypeStruct((N_NODES, IN_DIM), jnp.float32),
    )(feats, hist)


def _layer_body(edge_split, a0_ref, a1_ref, hdst_ref, hsrc_ref, w_ref, b_ref,
                o0_ref, o1_ref):
    if edge_split:
        a = a0_ref[...] + a1_ref[...]
    else:
        a = jnp.concatenate([a0_ref[...], a1_ref[...]], axis=1)
    a = a * lax.rsqrt(jnp.maximum(hdst_ref[:, 0:1], 1.0))
    y = jnp.dot(a, w_ref[...], preferred_element_type=jnp.float32) + b_ref[...]
    y = jnp.maximum(y, 0.0) * lax.rsqrt(jnp.maximum(hsrc_ref[:, 0:1], 1.0))
    o0_ref[...] = y[:, :IN_DIM]
    o1_ref[...] = y[:, IN_DIM:]


def _layer(agg, hist, w, b, edge_split):
    in_dim = w.shape[0]
    k = pl.pallas_call(
        functools.partial(_layer_body, edge_split),
        grid=(_GRID,),
        in_specs=[
            pl.BlockSpec((_ROWS_BLK, IN_DIM), lambda i: (i, 0)),
            pl.BlockSpec((_ROWS_BLK, IN_DIM), lambda i: (i + _GRID, 0)),
            pl.BlockSpec((_ROWS_BLK, 16), lambda i: (i + _GRID, 0)),
            pl.BlockSpec((_ROWS_BLK, 16), lambda i: (i, 0)),
            pl.BlockSpec((in_dim, HID), lambda i: (0, 0)),
            pl.BlockSpec((1, HID), lambda i: (0, 0)),
        ],
        out_specs=[
            pl.BlockSpec((_ROWS_BLK, IN_DIM), lambda i: (i, 0)),
            pl.BlockSpec((_ROWS_BLK, IN_DIM), lambda i: (i, 0)),
        ],
        out_shape=[
            jax.ShapeDtypeStruct((N_NODES, IN_DIM), jnp.float32),
            jax.ShapeDtypeStruct((N_NODES, IN_DIM), jnp.float32),
        ],
    )
    return k(agg, agg, hist, hist, w, b.reshape(1, HID))


def _final_body(a0_ref, a1_ref, hdst_ref, w_ref, b_ref, out_ref):
    i = pl.program_id(0)
    a = jnp.concatenate([a0_ref[...], a1_ref[...]], axis=1)
    a = a * lax.rsqrt(jnp.maximum(hdst_ref[:, 0:1], 1.0))
    y = jnp.dot(a, w_ref[...], preferred_element_type=jnp.float32) + b_ref[...]
    y = jnp.maximum(y, 0.0)
    blk = jnp.sum(y, axis=0, keepdims=True) * (1.0 / N_NODES)

    @pl.when(i == 0)
    def _():
        out_ref[...] = blk

    @pl.when(i > 0)
    def _():
        out_ref[...] += blk


def _final(agg, hist, w, b):
    return pl.pallas_call(
        _final_body,
        grid=(_GRID,),
        in_specs=[
            pl.BlockSpec((_ROWS_BLK, IN_DIM), lambda i: (i, 0)),
            pl.BlockSpec((_ROWS_BLK, IN_DIM), lambda i: (i + _GRID, 0)),
            pl.BlockSpec((_ROWS_BLK, 16), lambda i: (i + _GRID, 0)),
            pl.BlockSpec((HID, HID), lambda i: (0, 0)),
            pl.BlockSpec((1, HID), lambda i: (0, 0)),
        ],
        out_specs=pl.BlockSpec((1, HID), lambda i: (0, 0)),
        out_shape=jax.ShapeDtypeStruct((1, HID), jnp.float32),
    )(agg, agg, hist, w, b.reshape(1, HID))


def kernel(feats, edge_index, W0, b0, W1, b1, W2, b2):
    src = edge_index[0].astype(jnp.int32)
    dst = edge_index[1].astype(jnp.int32)

    hist = _hist(src, dst)
    x0 = _prep(feats, hist)
    agg0 = _spmm(x0, x0, src, dst, edge_split=True)
    h1a, h1b = _layer(agg0, hist, W0, b0, edge_split=True)
    agg1 = _spmm(h1a, h1b, src, dst, edge_split=False)
    h2a, h2b = _layer(agg1, hist, W1, b1, edge_split=False)
    agg2 = _spmm(h2a, h2b, src, dst, edge_split=False)
    return _final(agg2, hist, W2, b2)


# speedup vs baseline: 9.3499x; 1.2193x over previous
import functools

import jax
import jax.numpy as jnp
from jax import lax
from jax.experimental import pallas as pl
from jax.experimental.pallas import tpu as pltpu
from jax.experimental.pallas import tpu_sc as plsc

N_NODES = 10000
N_SLAB = 10240
N_EDGES = 320000
IN_DIM = 128
HID = 256

NC = 2
NS = 16
NW = 10
ROWS_PER_W = N_NODES // NW
CHUNK = 80
NBLK = 2000

_MESH = plsc.VectorSubcoreMesh(core_axis_name="c", subcore_axis_name="s")
_SC_PARAMS = pltpu.CompilerParams(use_tc_tiling_on_sc=False)


def _hist_body(src_hbm, dst_hbm, ones_hbm, zeros_hbm, out_hbm,
               idx_v, ones_v, zbuf_v, slab):
    c = lax.axis_index("c")
    s = lax.axis_index("s")

    pltpu.sync_copy(ones_hbm, ones_v)

    @pl.when(s < NW)
    def _():
        pltpu.sync_copy(zeros_hbm, zbuf_v)
        for k in range(5):
            pltpu.sync_copy(zbuf_v, slab.at[pl.ds(s * ROWS_PER_W + k * 200, 200)])

    plsc.subcore_barrier()

    n_e = N_EDGES // NS

    def run(e_ref):
        @pl.loop(0, n_e, step=NBLK)
        def _(i):
            pltpu.sync_copy(e_ref.at[pl.ds(s * n_e + i, NBLK)], idx_v)

            @pl.loop(0, NBLK, step=CHUNK)
            def _(j):
                pltpu.sync_copy(ones_v, slab.at[idx_v.at[pl.ds(j, CHUNK)]],
                                add=True)

    @pl.when(c == 0)
    def _():
        run(src_hbm)

    @pl.when(c == 1)
    def _():
        run(dst_hbm)

    plsc.subcore_barrier()

    @pl.when(s < NW)
    def _():
        for k in range(5):
            pltpu.sync_copy(
                slab.at[pl.ds(s * ROWS_PER_W + k * 200, 200)],
                out_hbm.at[pl.ds(c * N_NODES + s * ROWS_PER_W + k * 200, 200)],
            )


def _hist(src, dst):
    k = pl.kernel(
        _hist_body,
        out_type=jax.ShapeDtypeStruct((2 * N_NODES, 16), jnp.float32),
        mesh=_MESH,
        compiler_params=_SC_PARAMS,
        scratch_types=[
            pltpu.VMEM((NBLK,), jnp.int32),
            pltpu.VMEM((CHUNK, 16), jnp.float32),
            pltpu.VMEM((200, 16), jnp.float32),
            pltpu.VMEM_SHARED((N_SLAB, 16), jnp.float32),
        ],
    )
    ones = jnp.ones((CHUNK, 16), jnp.float32)
    zeros = jnp.zeros((200, 16), jnp.float32)
    return k(src, dst, ones, zeros)


def _spmm_body(edge_split, t0, t1, src_hbm, dst_hbm, zeros_hbm, out_hbm,
               sidx, didx, r0, r1, r2, r3, g0, g1, g2, g3, s0, s1, s2, s3,
               slab):
    c = lax.axis_index("c")
    s = lax.axis_index("s")
    bufs = (r0, r1, r2, r3)
    gsem = (g0, g1, g2, g3)
    ssem = (s0, s1, s2, s3)

    if edge_split:
        n_e = N_EDGES // (NC * NS)
        base_e = c * (N_EDGES // 2) + s * n_e
    else:
        n_e = N_EDGES // NS
        base_e = s * n_e

    @pl.when(s < NW)
    def _():
        pltpu.sync_copy(zeros_hbm, r0)
        for k in range(12):
            pltpu.sync_copy(r0, slab.at[pl.ds(s * ROWS_PER_W + k * CHUNK, CHUNK)])
        pltpu.sync_copy(r0.at[pl.ds(0, 40)],
                        slab.at[pl.ds(s * ROWS_PER_W + 12 * CHUNK, 40)])

    plsc.subcore_barrier()

    NCH = NBLK // CHUNK
    NQ = NCH // 4

    def gidx(j):
        return sidx.at[pl.ds(j, CHUNK)]

    def widx(j):
        return didx.at[pl.ds(j, CHUNK)]

    def run(tbl):
        def wait_gather(t):
            pltpu.make_async_copy(tbl.at[pl.ds(0, CHUNK)], bufs[t], gsem[t]).wait()

        @pl.loop(0, n_e, step=NBLK)
        def _(i):
            pltpu.sync_copy(src_hbm.at[pl.ds(base_e + i, NBLK)], sidx)
            pltpu.sync_copy(dst_hbm.at[pl.ds(base_e + i, NBLK)], didx)

            for t in range(4):
                pltpu.async_copy(tbl.at[gidx(t * CHUNK)], bufs[t], gsem[t])

            @pl.loop(0, NQ - 1)
            def _(q):
                jc = q * 4 * CHUNK
                jn = jc + 4 * CHUNK
                sc = []
                for t in range(4):
                    wait_gather(t)
                    sc.append(pltpu.async_copy(
                        bufs[t], slab.at[widx(jc + t * CHUNK)], ssem[t],
                        add=True))
                for t in range(4):
                    sc[t].wait()
                    pltpu.async_copy(tbl.at[gidx(jn + t * CHUNK)], bufs[t],
                                     gsem[t])

            jl = (NQ - 1) * 4 * CHUNK
            for t in range(4):
                wait_gather(t)
                pltpu.sync_copy(bufs[t], slab.at[widx(jl + t * CHUNK)], add=True)
            jt = NQ * 4 * CHUNK
            pltpu.async_copy(tbl.at[gidx(jt)], r0, g0).wait()
            pltpu.sync_copy(r0, slab.at[widx(jt)], add=True)

    @pl.when(c == 0)
    def _():
        run(t0)

    @pl.when(c == 1)
    def _():
        run(t1)

    plsc.subcore_barrier()

    @pl.when(s < NW)
    def _():
        pltpu.sync_copy(
            slab.at[pl.ds(s * ROWS_PER_W, ROWS_PER_W)],
            out_hbm.at[pl.ds(c * N_NODES + s * ROWS_PER_W, ROWS_PER_W)],
        )


def _spmm(t0, t1, src, dst, edge_split):
    k = pl.kernel(
        functools.partial(_spmm_body, edge_split),
        out_type=jax.ShapeDtypeStruct((2 * N_NODES, IN_DIM), jnp.bfloat16),
        mesh=_MESH,
        compiler_params=_SC_PARAMS,
        scratch_types=[
            pltpu.VMEM((NBLK,), jnp.int32),
            pltpu.VMEM((NBLK,), jnp.int32),
            pltpu.VMEM((CHUNK, IN_DIM), jnp.bfloat16),
            pltpu.VMEM((CHUNK, IN_DIM), jnp.bfloat16),
            pltpu.VMEM((CHUNK, IN_DIM), jnp.bfloat16),
            pltpu.VMEM((CHUNK, IN_DIM), jnp.bfloat16),
            pltpu.SemaphoreType.DMA,
            pltpu.SemaphoreType.DMA,
            pltpu.SemaphoreType.DMA,
            pltpu.SemaphoreType.DMA,
            pltpu.SemaphoreType.DMA,
            pltpu.SemaphoreType.DMA,
            pltpu.SemaphoreType.DMA,
            pltpu.SemaphoreType.DMA,
            pltpu.VMEM_SHARED((N_SLAB, IN_DIM), jnp.bfloat16),
        ],
    )
    zeros = jnp.zeros((CHUNK, IN_DIM), jnp.bfloat16)
    return k(t0, t1, src, dst, zeros)


_ROWS_BLK = 1000
_GRID = N_NODES // _ROWS_BLK


def _prep_body(feats_ref, hsrc_ref, x0_ref):
    dn = lax.rsqrt(jnp.maximum(hsrc_ref[:, 0:1], 1.0))
    x0_ref[...] = (feats_ref[...] * dn).astype(jnp.bfloat16)


def _prep(feats, hist):
    return pl.pallas_call(
        _prep_body,
        grid=(_GRID,),
        in_specs=[
            pl.BlockSpec((_ROWS_BLK, IN_DIM), lambda i: (i, 0)),
            pl.BlockSpec((_ROWS_BLK, 16), lambda i: (i, 0)),
        ],
        out_specs=pl.BlockSpec((_ROWS_BLK, IN_DIM), lambda i: (i, 0)),
        out_shape=jax.ShapeDtypeStruct((N_NODES, IN_DIM), jnp.bfloat16),
    )(feats, hist)


def _layer_body(edge_split, a0_ref, a1_ref, hdst_ref, hsrc_ref, w_ref, b_ref,
                o0_ref, o1_ref):
    if edge_split:
        a = a0_ref[...].astype(jnp.float32) + a1_ref[...].astype(jnp.float32)
    else:
        a = jnp.concatenate([a0_ref[...], a1_ref[...]], axis=1).astype(jnp.float32)
    a = a * lax.rsqrt(jnp.maximum(hdst_ref[:, 0:1], 1.0))
    y = jnp.dot(a, w_ref[...], preferred_element_type=jnp.float32) + b_ref[...]
    y = jnp.maximum(y, 0.0) * lax.rsqrt(jnp.maximum(hsrc_ref[:, 0:1], 1.0))
    y = y.astype(jnp.bfloat16)
    o0_ref[...] = y[:, :IN_DIM]
    o1_ref[...] = y[:, IN_DIM:]


def _layer(agg, hist, w, b, edge_split):
    in_dim = w.shape[0]
    k = pl.pallas_call(
        functools.partial(_layer_body, edge_split),
        grid=(_GRID,),
        in_specs=[
            pl.BlockSpec((_ROWS_BLK, IN_DIM), lambda i: (i, 0)),
            pl.BlockSpec((_ROWS_BLK, IN_DIM), lambda i: (i + _GRID, 0)),
            pl.BlockSpec((_ROWS_BLK, 16), lambda i: (i + _GRID, 0)),
            pl.BlockSpec((_ROWS_BLK, 16), lambda i: (i, 0)),
            pl.BlockSpec((in_dim, HID), lambda i: (0, 0)),
            pl.BlockSpec((1, HID), lambda i: (0, 0)),
        ],
        out_specs=[
            pl.BlockSpec((_ROWS_BLK, IN_DIM), lambda i: (i, 0)),
            pl.BlockSpec((_ROWS_BLK, IN_DIM), lambda i: (i, 0)),
        ],
        out_shape=[
            jax.ShapeDtypeStruct((N_NODES, IN_DIM), jnp.bfloat16),
            jax.ShapeDtypeStruct((N_NODES, IN_DIM), jnp.bfloat16),
        ],
    )
    return k(agg, agg, hist, hist, w, b.reshape(1, HID))


def _final_body(a0_ref, a1_ref, hdst_ref, w_ref, b_ref, out_ref):
    i = pl.program_id(0)
    a = jnp.concatenate([a0_ref[...], a1_ref[...]], axis=1).astype(jnp.float32)
    a = a * lax.rsqrt(jnp.maximum(hdst_ref[:, 0:1], 1.0))
    y = jnp.dot(a, w_ref[...], preferred_element_type=jnp.float32) + b_ref[...]
    y = jnp.maximum(y, 0.0)
    blk = jnp.sum(y, axis=0, keepdims=True) * (1.0 / N_NODES)

    @pl.when(i == 0)
    def _():
        out_ref[...] = blk

    @pl.when(i > 0)
    def _():
        out_ref[...] += blk


def _final(agg, hist, w, b):
    return pl.pallas_call(
        _final_body,
        grid=(_GRID,),
        in_specs=[
            pl.BlockSpec((_ROWS_BLK, IN_DIM), lambda i: (i, 0)),
            pl.BlockSpec((_ROWS_BLK, IN_DIM), lambda i: (i + _GRID, 0)),
            pl.BlockSpec((_ROWS_BLK, 16), lambda i: (i + _GRID, 0)),
            pl.BlockSpec((HID, HID), lambda i: (0, 0)),
            pl.BlockSpec((1, HID), lambda i: (0, 0)),
        ],
        out_specs=pl.BlockSpec((1, HID), lambda i: (0, 0)),
        out_shape=jax.ShapeDtypeStruct((1, HID), jnp.float32),
    )(agg, agg, hist, w, b.reshape(1, HID))


def kernel(feats, edge_index, W0, b0, W1, b1, W2, b2):
    src = edge_index[0].astype(jnp.int32)
    dst = edge_index[1].astype(jnp.int32)

    hist = _hist(src, dst)
    x0 = _prep(feats, hist)
    agg0 = _spmm(x0, x0, src, dst, edge_split=True)
    h1a, h1b = _layer(agg0, hist, W0, b0, edge_split=True)
    agg1 = _spmm(h1a, h1b, src, dst, edge_split=False)
    h2a, h2b = _layer(agg1, hist, W1, b1, edge_split=False)
    agg2 = _spmm(h2a, h2b, src, dst, edge_split=False)
    return _final(agg2, hist, W2, b2)
